# baseline (speedup 1.0000x reference)
import jax
import jax.numpy as jnp
from jax.experimental import pallas as pl

SIZE = 1024
LATENT_DIM = 64
BETA_C = 0.25
TOP_K = 3

N_TOTAL = 16 * 576
BLOCK_R = 288
NUM_BLOCKS = N_TOTAL // BLOCK_R


def _vq_kernel(z_ref, w_ref, rowsq_ref, wsq_ref, enc_ref, idx_ref, zq_ref,
               loss_ref, counts_ref, perp_ref):
    pid = pl.program_id(0)

    zb = z_ref[...]
    w = w_ref[...]

    zw = jax.lax.dot_general(
        zb, w, dimension_numbers=(((1,), (1,)), ((), ())),
        preferred_element_type=jnp.float32)
    d = rowsq_ref[...] + wsq_ref[...] - 2.0 * zw

    lane = jax.lax.broadcasted_iota(jnp.int32, (BLOCK_R, SIZE), 1)

    esum = jnp.zeros((BLOCK_R, SIZE), dtype=jnp.float32)
    for k in range(TOP_K):
        m = jnp.min(d, axis=1, keepdims=True)
        cand = jnp.where(d == m, lane, SIZE)
        idxk = jnp.min(cand, axis=1, keepdims=True)
        onehot = (lane == idxk).astype(jnp.float32)
        enc_ref[:, k, :] = onehot
        idx_ref[:, k:k + 1] = idxk
        esum = esum + onehot
        if k < TOP_K - 1:
            d = jnp.where(lane == idxk, jnp.inf, d)

    zq = jax.lax.dot_general(
        esum, w, dimension_numbers=(((1,), (0,)), ((), ())),
        preferred_element_type=jnp.float32) * (1.0 / TOP_K)
    zq_ref[...] = zq

    diff = zq - zb
    part_loss = jnp.sum(diff * diff)
    part_counts = jax.lax.dot_general(
        jnp.ones((1, BLOCK_R), dtype=jnp.float32), esum,
        dimension_numbers=(((1,), (0,)), ((), ())),
        preferred_element_type=jnp.float32)

    @pl.when(pid == 0)
    def _init():
        loss_ref[...] = jnp.zeros_like(loss_ref)
        counts_ref[...] = jnp.zeros_like(counts_ref)
        perp_ref[...] = jnp.zeros_like(perp_ref)

    loss_ref[...] += part_loss.reshape(1, 1)
    counts_ref[...] += part_counts

    @pl.when(pid == NUM_BLOCKS - 1)
    def _finish():
        total_sq = loss_ref[...]
        loss_ref[...] = (1.0 + BETA_C) * total_sq / (N_TOTAL * LATENT_DIM)
        e_mean = counts_ref[...] * (1.0 / (N_TOTAL * TOP_K))
        ent = jnp.sum(e_mean * jnp.log(e_mean + 1e-10))
        perp_ref[...] = jnp.exp(-ent).reshape(1, 1)


@jax.jit
def _vq_call(zf, W):
    rowsq = jnp.sum(zf ** 2, axis=1, keepdims=True)
    wsq = jnp.sum(W ** 2, axis=1)[None, :]
    grid = (NUM_BLOCKS,)
    out = pl.pallas_call(
        _vq_kernel,
        grid=grid,
        in_specs=[
            pl.BlockSpec((BLOCK_R, LATENT_DIM), lambda i: (i, 0)),
            pl.BlockSpec((SIZE, LATENT_DIM), lambda i: (0, 0)),
            pl.BlockSpec((BLOCK_R, 1), lambda i: (i, 0)),
            pl.BlockSpec((1, SIZE), lambda i: (0, 0)),
        ],
        out_specs=[
            pl.BlockSpec((BLOCK_R, TOP_K, SIZE), lambda i: (i, 0, 0)),
            pl.BlockSpec((BLOCK_R, TOP_K), lambda i: (i, 0)),
            pl.BlockSpec((BLOCK_R, LATENT_DIM), lambda i: (i, 0)),
            pl.BlockSpec((1, 1), lambda i: (0, 0)),
            pl.BlockSpec((1, SIZE), lambda i: (0, 0)),
            pl.BlockSpec((1, 1), lambda i: (0, 0)),
        ],
        out_shape=[
            jax.ShapeDtypeStruct((N_TOTAL, TOP_K, SIZE), jnp.float32),
            jax.ShapeDtypeStruct((N_TOTAL, TOP_K), jnp.int32),
            jax.ShapeDtypeStruct((N_TOTAL, LATENT_DIM), jnp.float32),
            jax.ShapeDtypeStruct((1, 1), jnp.float32),
            jax.ShapeDtypeStruct((1, SIZE), jnp.float32),
            jax.ShapeDtypeStruct((1, 1), jnp.float32),
        ],
    )(zf, W, rowsq, wsq)
    return out


def kernel(z, W):
    zf = z.reshape(-1, LATENT_DIM)
    enc, idx, zq, loss, _counts, perp = _vq_call(zf, W)
    z_q = zq.reshape(z.shape)
    z_q = z + jax.lax.stop_gradient(z_q - z)
    return (z_q, loss[0, 0], (perp[0, 0], enc, idx))


# speedup vs baseline: 1.0125x; 1.0125x over previous
import jax
import jax.numpy as jnp
from jax.experimental import pallas as pl
from jax.experimental.pallas import tpu as pltpu

SIZE = 1024
LATENT_DIM = 64
BETA_C = 0.25
TOP_K = 3

N_TOTAL = 16 * 576
BLOCK_R = 1152
NUM_BLOCKS = N_TOTAL // BLOCK_R


def _vq_kernel(z_ref, w_ref, rowsq_ref, wsq_ref, enc_ref, idx_ref, zq_ref,
               loss_ref, counts_ref, perp_ref):
    pid = pl.program_id(0)

    zb = z_ref[...]
    w = w_ref[...]

    zw = jax.lax.dot_general(
        zb, w, dimension_numbers=(((1,), (1,)), ((), ())),
        preferred_element_type=jnp.float32)
    d = rowsq_ref[...] + wsq_ref[...] - 2.0 * zw

    lane = jax.lax.broadcasted_iota(jnp.int32, (BLOCK_R, SIZE), 1)

    esum = jnp.zeros((BLOCK_R, SIZE), dtype=jnp.float32)
    for k in range(TOP_K):
        m = jnp.min(d, axis=1, keepdims=True)
        cand = jnp.where(d == m, lane, SIZE)
        idxk = jnp.min(cand, axis=1, keepdims=True)
        onehot = (lane == idxk).astype(jnp.float32)
        enc_ref[:, k, :] = onehot
        idx_ref[:, k:k + 1] = idxk
        esum = esum + onehot
        if k < TOP_K - 1:
            d = jnp.where(lane == idxk, jnp.inf, d)

    zq = jax.lax.dot_general(
        esum, w, dimension_numbers=(((1,), (0,)), ((), ())),
        preferred_element_type=jnp.float32) * (1.0 / TOP_K)
    zq_ref[...] = zq

    diff = zq - zb
    part_loss = jnp.sum(diff * diff)
    part_counts = jax.lax.dot_general(
        jnp.ones((1, BLOCK_R), dtype=jnp.float32), esum,
        dimension_numbers=(((1,), (0,)), ((), ())),
        preferred_element_type=jnp.float32)

    @pl.when(pid == 0)
    def _init():
        loss_ref[...] = jnp.zeros_like(loss_ref)
        counts_ref[...] = jnp.zeros_like(counts_ref)
        perp_ref[...] = jnp.zeros_like(perp_ref)

    loss_ref[...] += part_loss.reshape(1, 1)
    counts_ref[...] += part_counts

    @pl.when(pid == NUM_BLOCKS - 1)
    def _finish():
        total_sq = loss_ref[...]
        loss_ref[...] = (1.0 + BETA_C) * total_sq / (N_TOTAL * LATENT_DIM)
        e_mean = counts_ref[...] * (1.0 / (N_TOTAL * TOP_K))
        ent = jnp.sum(e_mean * jnp.log(e_mean + 1e-10))
        perp_ref[...] = jnp.exp(-ent).reshape(1, 1)


@jax.jit
def _vq_call(zf, W):
    rowsq = jnp.sum(zf ** 2, axis=1, keepdims=True)
    wsq = jnp.sum(W ** 2, axis=1)[None, :]
    grid = (NUM_BLOCKS,)
    out = pl.pallas_call(
        _vq_kernel,
        grid=grid,
        in_specs=[
            pl.BlockSpec((BLOCK_R, LATENT_DIM), lambda i: (i, 0)),
            pl.BlockSpec((SIZE, LATENT_DIM), lambda i: (0, 0)),
            pl.BlockSpec((BLOCK_R, 1), lambda i: (i, 0)),
            pl.BlockSpec((1, SIZE), lambda i: (0, 0)),
        ],
        out_specs=[
            pl.BlockSpec((BLOCK_R, TOP_K, SIZE), lambda i: (i, 0, 0)),
            pl.BlockSpec((BLOCK_R, TOP_K), lambda i: (i, 0)),
            pl.BlockSpec((BLOCK_R, LATENT_DIM), lambda i: (i, 0)),
            pl.BlockSpec((1, 1), lambda i: (0, 0)),
            pl.BlockSpec((1, SIZE), lambda i: (0, 0)),
            pl.BlockSpec((1, 1), lambda i: (0, 0)),
        ],
        out_shape=[
            jax.ShapeDtypeStruct((N_TOTAL, TOP_K, SIZE), jnp.float32),
            jax.ShapeDtypeStruct((N_TOTAL, TOP_K), jnp.int32),
            jax.ShapeDtypeStruct((N_TOTAL, LATENT_DIM), jnp.float32),
            jax.ShapeDtypeStruct((1, 1), jnp.float32),
            jax.ShapeDtypeStruct((1, SIZE), jnp.float32),
            jax.ShapeDtypeStruct((1, 1), jnp.float32),
        ],
        compiler_params=pltpu.CompilerParams(
            vmem_limit_bytes=110 * 1024 * 1024),
    )(zf, W, rowsq, wsq)
    return out


def kernel(z, W):
    zf = z.reshape(-1, LATENT_DIM)
    enc, idx, zq, loss, _counts, perp = _vq_call(zf, W)
    z_q = zq.reshape(z.shape)
    z_q = z + jax.lax.stop_gradient(z_q - z)
    return (z_q, loss[0, 0], (perp[0, 0], enc, idx))


# speedup vs baseline: 1.0373x; 1.0245x over previous
import jax
import jax.numpy as jnp
from jax.experimental import pallas as pl

SIZE = 1024
LATENT_DIM = 64
BETA_C = 0.25
TOP_K = 3

N_TOTAL = 16 * 576
BLOCK_R = 576
NUM_BLOCKS = N_TOTAL // BLOCK_R


def _vq_kernel(z_ref, w_ref, rowsq_ref, wsq_ref, enc_ref, idx_ref, zq_ref,
               loss_ref, counts_ref, perp_ref):
    pid = pl.program_id(0)

    zb = z_ref[...]
    w = w_ref[...]

    zw = jax.lax.dot_general(
        zb, w, dimension_numbers=(((1,), (1,)), ((), ())),
        preferred_element_type=jnp.float32)
    d = rowsq_ref[...] + wsq_ref[...] - 2.0 * zw

    lane = jax.lax.broadcasted_iota(jnp.int32, (BLOCK_R, SIZE), 1)

    esum = jnp.zeros((BLOCK_R, SIZE), dtype=jnp.float32)
    for k in range(TOP_K):
        m = jnp.min(d, axis=1, keepdims=True)
        cand = jnp.where(d == m, lane, SIZE)
        idxk = jnp.min(cand, axis=1, keepdims=True)
        onehot = (lane == idxk).astype(jnp.float32)
        enc_ref[:, k, :] = onehot
        idx_ref[:, k:k + 1] = idxk
        esum = esum + onehot
        if k < TOP_K - 1:
            d = jnp.where(lane == idxk, jnp.inf, d)

    zq = jax.lax.dot_general(
        esum, w, dimension_numbers=(((1,), (0,)), ((), ())),
        preferred_element_type=jnp.float32) * (1.0 / TOP_K)
    zq_ref[...] = zq

    diff = zq - zb
    part_loss = jnp.sum(diff * diff)
    part_counts = jax.lax.dot_general(
        jnp.ones((1, BLOCK_R), dtype=jnp.float32), esum,
        dimension_numbers=(((1,), (0,)), ((), ())),
        preferred_element_type=jnp.float32)

    @pl.when(pid == 0)
    def _init():
        loss_ref[...] = jnp.zeros_like(loss_ref)
        counts_ref[...] = jnp.zeros_like(counts_ref)
        perp_ref[...] = jnp.zeros_like(perp_ref)

    loss_ref[...] += part_loss.reshape(1, 1)
    counts_ref[...] += part_counts

    @pl.when(pid == NUM_BLOCKS - 1)
    def _finish():
        total_sq = loss_ref[...]
        loss_ref[...] = (1.0 + BETA_C) * total_sq / (N_TOTAL * LATENT_DIM)
        e_mean = counts_ref[...] * (1.0 / (N_TOTAL * TOP_K))
        ent = jnp.sum(e_mean * jnp.log(e_mean + 1e-10))
        perp_ref[...] = jnp.exp(-ent).reshape(1, 1)


@jax.jit
def _vq_call(zf, W):
    rowsq = jnp.sum(zf ** 2, axis=1, keepdims=True)
    wsq = jnp.sum(W ** 2, axis=1)[None, :]
    grid = (NUM_BLOCKS,)
    out = pl.pallas_call(
        _vq_kernel,
        grid=grid,
        in_specs=[
            pl.BlockSpec((BLOCK_R, LATENT_DIM), lambda i: (i, 0)),
            pl.BlockSpec((SIZE, LATENT_DIM), lambda i: (0, 0)),
            pl.BlockSpec((BLOCK_R, 1), lambda i: (i, 0)),
            pl.BlockSpec((1, SIZE), lambda i: (0, 0)),
        ],
        out_specs=[
            pl.BlockSpec((BLOCK_R, TOP_K, SIZE), lambda i: (i, 0, 0)),
            pl.BlockSpec((BLOCK_R, TOP_K), lambda i: (i, 0)),
            pl.BlockSpec((BLOCK_R, LATENT_DIM), lambda i: (i, 0)),
            pl.BlockSpec((1, 1), lambda i: (0, 0)),
            pl.BlockSpec((1, SIZE), lambda i: (0, 0)),
            pl.BlockSpec((1, 1), lambda i: (0, 0)),
        ],
        out_shape=[
            jax.ShapeDtypeStruct((N_TOTAL, TOP_K, SIZE), jnp.float32),
            jax.ShapeDtypeStruct((N_TOTAL, TOP_K), jnp.int32),
            jax.ShapeDtypeStruct((N_TOTAL, LATENT_DIM), jnp.float32),
            jax.ShapeDtypeStruct((1, 1), jnp.float32),
            jax.ShapeDtypeStruct((1, SIZE), jnp.float32),
            jax.ShapeDtypeStruct((1, 1), jnp.float32),
        ],
    )(zf, W, rowsq, wsq)
    return out


def kernel(z, W):
    zf = z.reshape(-1, LATENT_DIM)
    enc, idx, zq, loss, _counts, perp = _vq_call(zf, W)
    z_q = zq.reshape(z.shape)
    z_q = z + jax.lax.stop_gradient(z_q - z)
    return (z_q, loss[0, 0], (perp[0, 0], enc, idx))


# speedup vs baseline: 1.0462x; 1.0086x over previous
import jax
import jax.numpy as jnp
from jax.experimental import pallas as pl

SIZE = 1024
LATENT_DIM = 64
BETA_C = 0.25
TOP_K = 3

N_TOTAL = 16 * 576
BLOCK_R = 576
NUM_BLOCKS = N_TOTAL // BLOCK_R


def _vq_kernel(z_ref, w_ref, rowsq_ref, wsq_ref, enc_ref, idx_ref, zq_ref,
               loss_ref, counts_ref, perp_ref):
    pid = pl.program_id(0)

    zb = z_ref[...]
    w = w_ref[...]

    zw = jax.lax.dot_general(
        zb, w, dimension_numbers=(((1,), (1,)), ((), ())),
        preferred_element_type=jnp.float32)
    d = rowsq_ref[...] + wsq_ref[...] - 2.0 * zw

    lane = jax.lax.broadcasted_iota(jnp.int32, (BLOCK_R, SIZE), 1)

    esum = jnp.zeros((BLOCK_R, SIZE), dtype=jnp.float32)
    for k in range(TOP_K):
        m = jnp.min(d, axis=1, keepdims=True)
        cand = jnp.where(d == m, lane, SIZE)
        idxk = jnp.min(cand, axis=1, keepdims=True)
        onehot = (lane == idxk).astype(jnp.float32)
        enc_ref[:, k, :] = onehot
        idx_ref[:, k:k + 1] = idxk
        esum = esum + onehot
        if k < TOP_K - 1:
            d = jnp.where(lane == idxk, jnp.inf, d)

    zq = jax.lax.dot_general(
        esum, w, dimension_numbers=(((1,), (0,)), ((), ())),
        preferred_element_type=jnp.float32) * (1.0 / TOP_K)
    zq_ref[...] = zb + (zq - zb)

    diff = zq - zb
    part_loss = jnp.sum(diff * diff)
    part_counts = jax.lax.dot_general(
        jnp.ones((1, BLOCK_R), dtype=jnp.float32), esum,
        dimension_numbers=(((1,), (0,)), ((), ())),
        preferred_element_type=jnp.float32)

    @pl.when(pid == 0)
    def _init():
        loss_ref[...] = jnp.zeros_like(loss_ref)
        counts_ref[...] = jnp.zeros_like(counts_ref)
        perp_ref[...] = jnp.zeros_like(perp_ref)

    loss_ref[...] += part_loss.reshape(1, 1)
    counts_ref[...] += part_counts

    @pl.when(pid == NUM_BLOCKS - 1)
    def _finish():
        total_sq = loss_ref[...]
        loss_ref[...] = (1.0 + BETA_C) * total_sq / (N_TOTAL * LATENT_DIM)
        e_mean = counts_ref[...] * (1.0 / (N_TOTAL * TOP_K))
        ent = jnp.sum(e_mean * jnp.log(e_mean + 1e-10))
        perp_ref[...] = jnp.exp(-ent).reshape(1, 1)


@jax.jit
def _vq_call(zf, W):
    rowsq = jnp.sum(zf ** 2, axis=1, keepdims=True)
    wsq = jnp.sum(W ** 2, axis=1)[None, :]
    grid = (NUM_BLOCKS,)
    out = pl.pallas_call(
        _vq_kernel,
        grid=grid,
        in_specs=[
            pl.BlockSpec((BLOCK_R, LATENT_DIM), lambda i: (i, 0)),
            pl.BlockSpec((SIZE, LATENT_DIM), lambda i: (0, 0)),
            pl.BlockSpec((BLOCK_R, 1), lambda i: (i, 0)),
            pl.BlockSpec((1, SIZE), lambda i: (0, 0)),
        ],
        out_specs=[
            pl.BlockSpec((BLOCK_R, TOP_K, SIZE), lambda i: (i, 0, 0)),
            pl.BlockSpec((BLOCK_R, TOP_K), lambda i: (i, 0)),
            pl.BlockSpec((BLOCK_R, LATENT_DIM), lambda i: (i, 0)),
            pl.BlockSpec((1, 1), lambda i: (0, 0)),
            pl.BlockSpec((1, SIZE), lambda i: (0, 0)),
            pl.BlockSpec((1, 1), lambda i: (0, 0)),
        ],
        out_shape=[
            jax.ShapeDtypeStruct((N_TOTAL, TOP_K, SIZE), jnp.float32),
            jax.ShapeDtypeStruct((N_TOTAL, TOP_K), jnp.int32),
            jax.ShapeDtypeStruct((N_TOTAL, LATENT_DIM), jnp.float32),
            jax.ShapeDtypeStruct((1, 1), jnp.float32),
            jax.ShapeDtypeStruct((1, SIZE), jnp.float32),
            jax.ShapeDtypeStruct((1, 1), jnp.float32),
        ],
    )(zf, W, rowsq, wsq)
    return out


def kernel(z, W):
    zf = z.reshape(-1, LATENT_DIM)
    enc, idx, zq, loss, _counts, perp = _vq_call(zf, W)
    z_q = zq.reshape(z.shape)
    return (z_q, loss[0, 0], (perp[0, 0], enc, idx))


# speedup vs baseline: 1.0467x; 1.0005x over previous
import jax
import jax.numpy as jnp
from jax.experimental import pallas as pl
from jax.experimental.pallas import tpu as pltpu

SIZE = 1024
LATENT_DIM = 64
BETA_C = 0.25
TOP_K = 3

N_TOTAL = 16 * 576
BLOCK_R = 768
NUM_BLOCKS = N_TOTAL // BLOCK_R


def _vq_kernel(z_ref, w_ref, rowsq_ref, wsq_ref, enc_ref, idx_ref, zq_ref,
               loss_ref, counts_ref, perp_ref):
    pid = pl.program_id(0)

    zb = z_ref[...]
    w = w_ref[...]

    zw = jax.lax.dot_general(
        zb, w, dimension_numbers=(((1,), (1,)), ((), ())),
        preferred_element_type=jnp.float32)
    d = rowsq_ref[...] + wsq_ref[...] - 2.0 * zw

    lane = jax.lax.broadcasted_iota(jnp.int32, (BLOCK_R, SIZE), 1)

    esum = jnp.zeros((BLOCK_R, SIZE), dtype=jnp.float32)
    for k in range(TOP_K):
        m = jnp.min(d, axis=1, keepdims=True)
        cand = jnp.where(d == m, lane, SIZE)
        idxk = jnp.min(cand, axis=1, keepdims=True)
        onehot = (lane == idxk).astype(jnp.float32)
        enc_ref[:, k, :] = onehot
        idx_ref[:, k:k + 1] = idxk
        esum = esum + onehot
        if k < TOP_K - 1:
            d = jnp.where(lane == idxk, jnp.inf, d)

    zq = jax.lax.dot_general(
        esum, w, dimension_numbers=(((1,), (0,)), ((), ())),
        preferred_element_type=jnp.float32) * (1.0 / TOP_K)
    zq_ref[...] = zb + (zq - zb)

    diff = zq - zb
    part_loss = jnp.sum(diff * diff)
    part_counts = jax.lax.dot_general(
        jnp.ones((1, BLOCK_R), dtype=jnp.float32), esum,
        dimension_numbers=(((1,), (0,)), ((), ())),
        preferred_element_type=jnp.float32)

    @pl.when(pid == 0)
    def _init():
        loss_ref[...] = jnp.zeros_like(loss_ref)
        counts_ref[...] = jnp.zeros_like(counts_ref)
        perp_ref[...] = jnp.zeros_like(perp_ref)

    loss_ref[...] += part_loss.reshape(1, 1)
    counts_ref[...] += part_counts

    @pl.when(pid == NUM_BLOCKS - 1)
    def _finish():
        total_sq = loss_ref[...]
        loss_ref[...] = (1.0 + BETA_C) * total_sq / (N_TOTAL * LATENT_DIM)
        e_mean = counts_ref[...] * (1.0 / (N_TOTAL * TOP_K))
        ent = jnp.sum(e_mean * jnp.log(e_mean + 1e-10))
        perp_ref[...] = jnp.exp(-ent).reshape(1, 1)


@jax.jit
def _vq_call(zf, W):
    rowsq = jnp.sum(zf ** 2, axis=1, keepdims=True)
    wsq = jnp.sum(W ** 2, axis=1)[None, :]
    grid = (NUM_BLOCKS,)
    out = pl.pallas_call(
        _vq_kernel,
        grid=grid,
        in_specs=[
            pl.BlockSpec((BLOCK_R, LATENT_DIM), lambda i: (i, 0)),
            pl.BlockSpec((SIZE, LATENT_DIM), lambda i: (0, 0)),
            pl.BlockSpec((BLOCK_R, 1), lambda i: (i, 0)),
            pl.BlockSpec((1, SIZE), lambda i: (0, 0)),
        ],
        out_specs=[
            pl.BlockSpec((BLOCK_R, TOP_K, SIZE), lambda i: (i, 0, 0)),
            pl.BlockSpec((BLOCK_R, TOP_K), lambda i: (i, 0)),
            pl.BlockSpec((BLOCK_R, LATENT_DIM), lambda i: (i, 0)),
            pl.BlockSpec((1, 1), lambda i: (0, 0)),
            pl.BlockSpec((1, SIZE), lambda i: (0, 0)),
            pl.BlockSpec((1, 1), lambda i: (0, 0)),
        ],
        out_shape=[
            jax.ShapeDtypeStruct((N_TOTAL, TOP_K, SIZE), jnp.float32),
            jax.ShapeDtypeStruct((N_TOTAL, TOP_K), jnp.int32),
            jax.ShapeDtypeStruct((N_TOTAL, LATENT_DIM), jnp.float32),
            jax.ShapeDtypeStruct((1, 1), jnp.float32),
            jax.ShapeDtypeStruct((1, SIZE), jnp.float32),
            jax.ShapeDtypeStruct((1, 1), jnp.float32),
        ],
        compiler_params=pltpu.CompilerParams(
            vmem_limit_bytes=110 * 1024 * 1024),
    )(zf, W, rowsq, wsq)
    return out


def kernel(z, W):
    zf = z.reshape(-1, LATENT_DIM)
    enc, idx, zq, loss, _counts, perp = _vq_call(zf, W)
    z_q = zq.reshape(z.shape)
    return (z_q, loss[0, 0], (perp[0, 0], enc, idx))


# speedup vs baseline: 1.0468x; 1.0001x over previous
import jax
import jax.numpy as jnp
from jax.experimental import pallas as pl

SIZE = 1024
LATENT_DIM = 64
BETA_C = 0.25
TOP_K = 3

N_TOTAL = 16 * 576
BLOCK_R = 576
NUM_BLOCKS = N_TOTAL // BLOCK_R


def _vq_kernel(z_ref, w_ref, rowsq_ref, wsq_ref, enc_ref, idx_ref, zq_ref,
               loss_ref, counts_ref, perp_ref):
    pid = pl.program_id(0)

    zb = z_ref[...]
    w = w_ref[...]

    zw = jax.lax.dot_general(
        zb, w, dimension_numbers=(((1,), (1,)), ((), ())),
        preferred_element_type=jnp.float32)
    d = rowsq_ref[...] + wsq_ref[...] - 2.0 * zw

    lane = jax.lax.broadcasted_iota(jnp.int32, (BLOCK_R, SIZE), 1)

    esum = jnp.zeros((BLOCK_R, SIZE), dtype=jnp.float32)
    for k in range(TOP_K):
        m = jnp.min(d, axis=1, keepdims=True)
        cand = jnp.where(d == m, lane, SIZE)
        idxk = jnp.min(cand, axis=1, keepdims=True)
        onehot = (lane == idxk).astype(jnp.float32)
        enc_ref[:, k, :] = onehot
        idx_ref[:, k:k + 1] = idxk
        esum = esum + onehot
        if k < TOP_K - 1:
            d = jnp.where(lane == idxk, jnp.inf, d)

    zq = jax.lax.dot_general(
        esum, w, dimension_numbers=(((1,), (0,)), ((), ())),
        preferred_element_type=jnp.float32) * (1.0 / TOP_K)
    zq_ref[...] = zb + (zq - zb)

    diff = zq - zb
    part_loss = jnp.sum(diff * diff)
    part_counts = jax.lax.dot_general(
        jnp.ones((1, BLOCK_R), dtype=jnp.float32), esum,
        dimension_numbers=(((1,), (0,)), ((), ())),
        preferred_element_type=jnp.float32)

    @pl.when(pid == 0)
    def _init():
        loss_ref[...] = jnp.zeros_like(loss_ref)
        counts_ref[...] = jnp.zeros_like(counts_ref)
        perp_ref[...] = jnp.zeros_like(perp_ref)

    loss_ref[...] += part_loss.reshape(1, 1)
    counts_ref[...] += part_counts

    @pl.when(pid == NUM_BLOCKS - 1)
    def _finish():
        total_sq = loss_ref[...]
        loss_ref[...] = (1.0 + BETA_C) * total_sq / (N_TOTAL * LATENT_DIM)
        e_mean = counts_ref[...] * (1.0 / (N_TOTAL * TOP_K))
        ent = jnp.sum(e_mean * jnp.log(e_mean + 1e-10))
        perp_ref[...] = jnp.exp(-ent).reshape(1, 1)


@jax.jit
def _vq_call(zf, W):
    rowsq = jnp.sum(zf ** 2, axis=1, keepdims=True)
    wsq = jnp.sum(W ** 2, axis=1)[None, :]
    grid = (NUM_BLOCKS,)
    out = pl.pallas_call(
        _vq_kernel,
        grid=grid,
        in_specs=[
            pl.BlockSpec((BLOCK_R, LATENT_DIM), lambda i: (i, 0)),
            pl.BlockSpec((SIZE, LATENT_DIM), lambda i: (0, 0)),
            pl.BlockSpec((BLOCK_R, 1), lambda i: (i, 0)),
            pl.BlockSpec((1, SIZE), lambda i: (0, 0)),
        ],
        out_specs=[
            pl.BlockSpec((BLOCK_R, TOP_K, SIZE), lambda i: (i, 0, 0)),
            pl.BlockSpec((BLOCK_R, TOP_K), lambda i: (i, 0)),
            pl.BlockSpec((BLOCK_R, LATENT_DIM), lambda i: (i, 0)),
            pl.BlockSpec((1, 1), lambda i: (0, 0)),
            pl.BlockSpec((1, SIZE), lambda i: (0, 0)),
            pl.BlockSpec((1, 1), lambda i: (0, 0)),
        ],
        out_shape=[
            jax.ShapeDtypeStruct((N_TOTAL, TOP_K, SIZE), jnp.float32),
            jax.ShapeDtypeStruct((N_TOTAL, TOP_K), jnp.int32),
            jax.ShapeDtypeStruct((N_TOTAL, LATENT_DIM), jnp.float32),
            jax.ShapeDtypeStruct((1, 1), jnp.float32),
            jax.ShapeDtypeStruct((1, SIZE), jnp.float32),
            jax.ShapeDtypeStruct((1, 1), jnp.float32),
        ],
    )(zf, W, rowsq, wsq)
    return out


def kernel(z, W):
    zf = z.reshape(-1, LATENT_DIM)
    enc, idx, zq, loss, _counts, perp = _vq_call(zf, W)
    z_q = zq.reshape(z.shape)
    return (z_q, loss[0, 0], (perp[0, 0], enc, idx))
